# Initial kernel scaffold; baseline (speedup 1.0000x reference)
import jax
import jax.numpy as jnp
from jax.experimental import pallas as pl


def kernel(outputs_key, targets_mask_key, targets_ind_key, targets_key):
    raise NotImplementedError("write your pallas kernel here")


# speedup vs baseline: 2.2816x; 2.2816x over previous
import functools

import jax
import jax.numpy as jnp
from jax import lax
from jax.experimental import pallas as pl
from jax.experimental.pallas import tpu as pltpu
from jax.experimental.pallas import tpu_sc as plsc

B, C, H, W = 32, 64, 128, 128
HW = H * W
K = 500
KP = 512
NC, NS, L = 2, 16, 16
NCHUNK = KP // L
GW = 128
NG = KP // GW


def _sc_body(outs_hbm, ind_hbm, mask_hbm, tgt_hbm, out_hbm,
             ind_v, mask_v, idx_v, pred_v, tgt_v, res_v, sem, tsem):
    b = lax.axis_index("s") * NC + lax.axis_index("c")
    pltpu.sync_copy(ind_hbm.at[b], ind_v)
    pltpu.sync_copy(mask_hbm.at[b], mask_v)

    zero = jnp.zeros((L,), jnp.float32)
    cnt = zero
    for j in range(NCHUNK):
        cnt = cnt + mask_v[pl.ds(j * L, L)].astype(jnp.float32)

    def c_step(c, acc):
        base = (b * C + c) * HW
        for j in range(NCHUNK):
            r, t = divmod(j * L, GW)
            idx_v[r, pl.ds(t, L)] = ind_v[pl.ds(j * L, L)] + base
        tcopy = pltpu.make_async_copy(tgt_hbm.at[b, c], tgt_v, tsem)
        tcopy.start()
        copies = []
        for r in range(NG):
            cp = pltpu.make_async_copy(
                outs_hbm.at[idx_v.at[r]], pred_v.at[pl.ds(r * GW, GW)], sem)
            cp.start()
            copies.append(cp)
        tcopy.wait()
        for cp in copies:
            cp.wait()
        for j in range(NCHUNK):
            pv = pred_v[pl.ds(j * L, L)]
            tv = tgt_v[pl.ds(j * L, L)]
            mv = mask_v[pl.ds(j * L, L)]
            acc = acc + jnp.where(mv > 0, jnp.abs(pv - tv), 0.0)
        return acc

    acc = lax.fori_loop(0, C, c_step, zero)
    res_v[0, :] = acc
    res_v[1, :] = cnt
    pltpu.sync_copy(res_v, out_hbm.at[b])


@jax.jit
def kernel(outputs_key, targets_mask_key, targets_ind_key, targets_key):
    outs_flat = outputs_key.reshape(B * C * HW)
    ind_p = jnp.pad(targets_ind_key, ((0, 0), (0, KP - K)))
    mask_p = jnp.pad(targets_mask_key, ((0, 0), (0, KP - K)))
    tgt_t = jnp.pad(jnp.transpose(targets_key, (0, 2, 1)),
                    ((0, 0), (0, 0), (0, KP - K)))

    mesh = plsc.VectorSubcoreMesh(core_axis_name="c", subcore_axis_name="s")
    f = pl.kernel(
        _sc_body,
        out_type=jax.ShapeDtypeStruct((B, 2, L), jnp.float32),
        mesh=mesh,
        scratch_types=[
            pltpu.VMEM((KP,), jnp.int32),
            pltpu.VMEM((KP,), jnp.int32),
            pltpu.VMEM((NG, GW), jnp.int32),
            pltpu.VMEM((KP,), jnp.float32),
            pltpu.VMEM((KP,), jnp.float32),
            pltpu.VMEM((2, L), jnp.float32),
            pltpu.SemaphoreType.DMA,
            pltpu.SemaphoreType.DMA,
        ],
    )
    part = f(outs_flat, ind_p, mask_p, tgt_t)
    num = jnp.sum(part[:, 0, :])
    cnt = jnp.sum(part[:, 1, :])
    loss = num / (B * K * C)
    return loss / (C * cnt + 0.0001)


# speedup vs baseline: 3.4010x; 1.4906x over previous
import functools

import jax
import jax.numpy as jnp
from jax import lax
from jax.experimental import pallas as pl
from jax.experimental.pallas import tpu as pltpu
from jax.experimental.pallas import tpu_sc as plsc

B, C, H, W = 32, 64, 128, 128
HW = H * W
K = 500
KP = 512
NC, NS, L = 2, 16, 16
NCHUNK = KP // L
GW = 128
NG = KP // GW


def _sc_body(outs_hbm, ind_hbm, mask_hbm, tgt_hbm, out_hbm,
             ind_v, mask_v, idx0, idx1, pred0, pred1, tgt0, tgt1, res_v,
             sem0, sem1, ts0, ts1):
    b = lax.axis_index("s") * NC + lax.axis_index("c")
    pltpu.sync_copy(ind_hbm.at[b], ind_v)
    pltpu.sync_copy(mask_hbm.at[b], mask_v)

    bufs = ((idx0, pred0, tgt0, sem0, ts0), (idx1, pred1, tgt1, sem1, ts1))

    def fire(c, buf):
        idx_v, pred_v, tgt_v, sem, tsem = buf
        base = (b * C + c) * HW
        for j in range(NCHUNK):
            r, t = divmod(j * L, GW)
            idx_v[r, pl.ds(t, L)] = ind_v[pl.ds(j * L, L)] + base
        pltpu.make_async_copy(tgt_hbm.at[b, c], tgt_v, tsem).start()
        for r in range(NG):
            pltpu.make_async_copy(
                outs_hbm.at[idx_v.at[r]], pred_v.at[pl.ds(r * GW, GW)],
                sem).start()

    def drain_accum(c, buf, acc):
        idx_v, pred_v, tgt_v, sem, tsem = buf
        pltpu.make_async_copy(tgt_hbm.at[b, c], tgt_v, tsem).wait()
        for r in range(NG):
            pltpu.make_async_copy(
                outs_hbm.at[idx_v.at[r]], pred_v.at[pl.ds(r * GW, GW)],
                sem).wait()
        for j in range(NCHUNK):
            pv = pred_v[pl.ds(j * L, L)]
            tv = tgt_v[pl.ds(j * L, L)]
            mv = mask_v[pl.ds(j * L, L)]
            acc = acc + jnp.where(mv > 0, jnp.abs(pv - tv), 0.0)
        return acc

    zero = jnp.zeros((L,), jnp.float32)
    cnt = zero
    for j in range(NCHUNK):
        cnt = cnt + mask_v[pl.ds(j * L, L)].astype(jnp.float32)

    fire(0, bufs[0])

    def pair_step(i, acc):
        c0 = 2 * i
        fire(c0 + 1, bufs[1])
        acc = drain_accum(c0, bufs[0], acc)

        @pl.when(c0 + 2 < C)
        def _():
            fire(c0 + 2, bufs[0])

        return drain_accum(c0 + 1, bufs[1], acc)

    acc = lax.fori_loop(0, C // 2, pair_step, zero)
    res_v[0, :] = acc
    res_v[1, :] = cnt
    pltpu.sync_copy(res_v, out_hbm.at[b])


@jax.jit
def kernel(outputs_key, targets_mask_key, targets_ind_key, targets_key):
    outs_flat = outputs_key.reshape(B * C * HW)
    ind_p = jnp.pad(targets_ind_key, ((0, 0), (0, KP - K)))
    mask_p = jnp.pad(targets_mask_key, ((0, 0), (0, KP - K)))
    tgt_t = jnp.pad(jnp.transpose(targets_key, (0, 2, 1)),
                    ((0, 0), (0, 0), (0, KP - K)))

    mesh = plsc.VectorSubcoreMesh(core_axis_name="c", subcore_axis_name="s")
    f = pl.kernel(
        _sc_body,
        out_type=jax.ShapeDtypeStruct((B, 2, L), jnp.float32),
        mesh=mesh,
        scratch_types=[
            pltpu.VMEM((KP,), jnp.int32),
            pltpu.VMEM((KP,), jnp.int32),
            pltpu.VMEM((NG, GW), jnp.int32),
            pltpu.VMEM((NG, GW), jnp.int32),
            pltpu.VMEM((KP,), jnp.float32),
            pltpu.VMEM((KP,), jnp.float32),
            pltpu.VMEM((KP,), jnp.float32),
            pltpu.VMEM((KP,), jnp.float32),
            pltpu.VMEM((2, L), jnp.float32),
            pltpu.SemaphoreType.DMA,
            pltpu.SemaphoreType.DMA,
            pltpu.SemaphoreType.DMA,
            pltpu.SemaphoreType.DMA,
        ],
    )
    part = f(outs_flat, ind_p, mask_p, tgt_t)
    num = jnp.sum(part[:, 0, :])
    cnt = jnp.sum(part[:, 1, :])
    loss = num / (B * K * C)
    return loss / (C * cnt + 0.0001)
